# baseline (speedup 1.0000x reference)
import functools

import jax
import jax.numpy as jnp
from jax import lax
from jax.experimental import pallas as pl
from jax.experimental.pallas import tpu as pltpu
from jax.experimental.pallas import tpu_sc as plsc

_TPS = 2


def kernel(x):
    B, F, K, ED = x.shape
    G = F // _TPS
    frames = B * F
    NC, NS = 2, 16
    NW = NC * NS
    per_w = frames // NW

    xf = x.reshape(frames, K, 1, ED)

    mesh = plsc.VectorSubcoreMesh(core_axis_name="c", subcore_axis_name="s")

    @functools.partial(
        pl.kernel,
        out_type=jax.ShapeDtypeStruct((B * G, K, _TPS, ED), jnp.float32),
        mesh=mesh,
        scratch_types=[pltpu.SemaphoreType.DMA],
    )
    def merge(x_hbm, out_hbm, sem):
        wid = lax.axis_index("s") * NC + lax.axis_index("c")
        base = wid * per_w
        copies = []
        for j in range(per_w):
            frame = base + j
            g = frame // _TPS
            i = frame % _TPS
            copies.append(
                pltpu.async_copy(
                    x_hbm.at[frame], out_hbm.at[g, :, pl.ds(i, 1), :], sem
                )
            )
        for c in copies:
            c.wait()

    out = merge(xf)
    return out.reshape(B, G, K, _TPS * ED)


# speedup vs baseline: 5.6007x; 5.6007x over previous
import functools

import jax
import jax.numpy as jnp
from jax import lax
from jax.experimental import pallas as pl
from jax.experimental.pallas import tpu as pltpu
from jax.experimental.pallas import tpu_sc as plsc

_TPS = 2


def kernel(x):
    B, F, K, ED = x.shape
    G = F // _TPS
    frames = B * F
    NC, NS = 2, 16
    NW = NC * NS
    HK = K // 2
    chunks = frames
    per_w = chunks // NW

    xf = x.reshape(frames, K, 1, ED)

    mesh = plsc.VectorSubcoreMesh(core_axis_name="c", subcore_axis_name="s")

    @functools.partial(
        pl.kernel,
        out_type=jax.ShapeDtypeStruct((B * G, K, _TPS, ED), jnp.float32),
        mesh=mesh,
        scratch_types=[
            pltpu.VMEM((HK, _TPS, ED), jnp.float32),
            pltpu.VMEM((HK, _TPS, ED), jnp.float32),
            pltpu.SemaphoreType.DMA,
            pltpu.SemaphoreType.DMA,
            pltpu.SemaphoreType.DMA,
            pltpu.SemaphoreType.DMA,
        ],
    )
    def merge(x_hbm, out_hbm, buf0, buf1, isem0, isem1, osem0, osem1):
        wid = lax.axis_index("s") * NC + lax.axis_index("c")
        base = wid * per_w
        bufs = (buf0, buf1)
        isems = (isem0, isem1)
        osems = (osem0, osem1)
        outs = [None, None]
        for j in range(per_w):
            c = base + j
            g = c // 2
            k0 = (c % 2) * HK
            s = j % 2
            if outs[s] is not None:
                outs[s].wait()
            i0 = pltpu.async_copy(
                x_hbm.at[2 * g, pl.ds(k0, HK)],
                bufs[s].at[:, pl.ds(0, 1), :],
                isems[s],
            )
            i1 = pltpu.async_copy(
                x_hbm.at[2 * g + 1, pl.ds(k0, HK)],
                bufs[s].at[:, pl.ds(1, 1), :],
                isems[s],
            )
            i0.wait()
            i1.wait()
            outs[s] = pltpu.async_copy(
                bufs[s], out_hbm.at[g, pl.ds(k0, HK)], osems[s]
            )
        for o in outs:
            o.wait()

    out = merge(xf)
    return out.reshape(B, G, K, _TPS * ED)
